# baseline (speedup 1.0000x reference)
import functools

import jax
import jax.numpy as jnp
from jax import lax
from jax.experimental import pallas as pl
from jax.experimental.pallas import tpu as pltpu
from jax.experimental.pallas import tpu_sc as plsc

N = 10000
E = 160000
IN_CH = 256
OUT_CH = 1024
HEADS = 4
HEAD_DIM = OUT_CH // HEADS

NB = 512
GROWS = (N + NB - 1) // NB

C = 256
SHB = 8
SHL = 9
PASSES = 20
HALF = PASSES * C
NPAD = 2 * HALF
EPT = E // 16
CPAD = C + 16
ROW = OUT_CH + 128
EBLK = 1000
NBLK = EPT // EBLK
WCAP = EPT + PASSES * 32 + 32


def _proj_body(x_ref, w_ref, b_ref, we_ref, out_ref, qe_ref):
    o = (jnp.dot(x_ref[...], w_ref[...], preferred_element_type=jnp.float32)
         + b_ref[...])
    out_ref[...] = o
    qw = o[:, :OUT_CH] * we_ref[...]
    cols = [
        jnp.sum(qw[:, h * HEAD_DIM:(h + 1) * HEAD_DIM], axis=1,
                keepdims=True)
        for h in range(HEADS)
    ]
    cols.append(jnp.zeros((qw.shape[0], 128 - HEADS), jnp.float32))
    qe_ref[...] = jnp.concatenate(cols, axis=1)


def _projections(x, Wcat, bcat, we):
    return pl.pallas_call(
        _proj_body,
        grid=(GROWS,),
        in_specs=[
            pl.BlockSpec((NB, IN_CH), lambda i: (i, 0)),
            pl.BlockSpec((IN_CH, 4 * OUT_CH), lambda i: (0, 0)),
            pl.BlockSpec((1, 4 * OUT_CH), lambda i: (0, 0)),
            pl.BlockSpec((1, OUT_CH), lambda i: (0, 0)),
        ],
        out_specs=[
            pl.BlockSpec((NB, 4 * OUT_CH), lambda i: (i, 0)),
            pl.BlockSpec((NB, 128), lambda i: (i, 0)),
        ],
        out_shape=[
            jax.ShapeDtypeStruct((N, 4 * OUT_CH), jnp.float32),
            jax.ShapeDtypeStruct((N, 128), jnp.float32),
        ],
    )(x, Wcat, bcat, we)


def _stats_body(att_ref, skip_ref, out_ref, s1_ref, s2_ref):
    i = pl.program_id(0)
    o = att_ref[...] + skip_ref[...]
    out_ref[...] = o
    row = lax.broadcasted_iota(jnp.int32, o.shape, 0) + i * NB
    om = jnp.where(row < N, o, 0.0)
    s1 = jnp.sum(om, axis=0, keepdims=True)
    s2 = jnp.sum(om * om, axis=0, keepdims=True)

    @pl.when(i == 0)
    def _init():
        s1_ref[...] = jnp.zeros_like(s1_ref)
        s2_ref[...] = jnp.zeros_like(s2_ref)

    s1_ref[...] += s1
    s2_ref[...] += s2


def _stats(att, skip):
    return pl.pallas_call(
        _stats_body,
        grid=(GROWS,),
        in_specs=[
            pl.BlockSpec((NB, OUT_CH), lambda i: (i, 0)),
            pl.BlockSpec((NB, OUT_CH), lambda i: (i, 0)),
        ],
        out_specs=[
            pl.BlockSpec((NB, OUT_CH), lambda i: (i, 0)),
            pl.BlockSpec((1, OUT_CH), lambda i: (0, 0)),
            pl.BlockSpec((1, OUT_CH), lambda i: (0, 0)),
        ],
        out_shape=[
            jax.ShapeDtypeStruct((N, OUT_CH), jnp.float32),
            jax.ShapeDtypeStruct((1, OUT_CH), jnp.float32),
            jax.ShapeDtypeStruct((1, OUT_CH), jnp.float32),
        ],
    )(att, skip)


def _gram_body(out_ref, s1_ref, s2_ref, gw_ref, gb_ref, gms_ref, gram_ref):
    i = pl.program_id(0)
    mean = s1_ref[...] / N
    ex2 = s2_ref[...] / N
    g = gms_ref[...]
    var = ex2 - (2.0 * g - g * g) * mean * mean
    inv = lax.rsqrt(var + 1e-5)
    o = out_ref[...]
    onorm = gw_ref[...] * (o - g * mean) * inv + gb_ref[...]
    onorm = jnp.maximum(onorm, 0.0)
    row = lax.broadcasted_iota(jnp.int32, onorm.shape, 0) + i * NB
    onorm = jnp.where(row < N, onorm, 0.0)

    @pl.when(i == 0)
    def _init():
        gram_ref[...] = jnp.zeros_like(gram_ref)

    gram_ref[...] += lax.dot_general(
        onorm, onorm, (((0,), (0,)), ((), ())),
        preferred_element_type=jnp.float32,
    )


def _gram(out, s1, s2, gw, gb, gms):
    return pl.pallas_call(
        _gram_body,
        grid=(GROWS,),
        in_specs=[
            pl.BlockSpec((NB, OUT_CH), lambda i: (i, 0)),
            pl.BlockSpec((1, OUT_CH), lambda i: (0, 0)),
            pl.BlockSpec((1, OUT_CH), lambda i: (0, 0)),
            pl.BlockSpec((1, OUT_CH), lambda i: (0, 0)),
            pl.BlockSpec((1, OUT_CH), lambda i: (0, 0)),
            pl.BlockSpec((1, OUT_CH), lambda i: (0, 0)),
        ],
        out_specs=pl.BlockSpec((OUT_CH, OUT_CH), lambda i: (0, 0)),
        out_shape=jax.ShapeDtypeStruct((OUT_CH, OUT_CH), jnp.float32),
    )(out, s1, s2, gw, gb, gms)


def _vsum16(v):
    vals = [v[t] for t in range(16)]
    while len(vals) > 1:
        nxt = [vals[i] + vals[i + 1] for i in range(0, len(vals) - 1, 2)]
        if len(vals) % 2:
            nxt.append(vals[-1])
        vals = nxt
    return vals[0]


def _edge_body(q_hbm, kv_hbm, src_hbm, dst_hbm, attr_hbm, we_hbm,
               att_hbm, dblk, sblk, ablk, wl_pack, wl_attr, segbuf, offbuf,
               cntb, qbuf, kvbuf, msg, qidx, srcidx, dstidx, abuf,
               webuf, rowbuf, zrow, numer_sh, semq, semk, semm):
    c = lax.axis_index("c")
    s = lax.axis_index("s")
    lane = lax.broadcasted_iota(jnp.int32, (16,), 0)
    base = c * HALF
    ebase = s * EPT
    z16 = jnp.zeros((16,), jnp.float32)

    pltpu.sync_copy(we_hbm, webuf)

    def _zcol(i, carry3):
        def _zc2(r, carry4):
            zrow[r, pl.ds(i * 16, 16)] = z16
            return carry4
        lax.fori_loop(0, 8, _zc2, 0)
        return carry3

    lax.fori_loop(0, 128 // 16, _zcol, 0)

    def _mfill(r, carry2):
        def _mcol(i, carry3):
            msg[8, r, pl.ds(32 + i * 16, 16)] = z16
            return carry3
        lax.fori_loop(0, (128 - 32) // 16, _mcol, 0)
        return carry2

    lax.fori_loop(0, 16, _mfill, 0)

    for k in range(9):
        pltpu.sync_copy(zrow, numer_sh.at[k, pl.ds(s * 17, 8)])
        pltpu.sync_copy(zrow, numer_sh.at[k, pl.ds(s * 17 + 8, 8)])
        pltpu.sync_copy(zrow.at[pl.ds(0, 1)],
                        numer_sh.at[k, pl.ds(s * 17 + 16, 1)])

    def _czero(bb, carry):
        cntb[pl.ds(bb * 16, 16)] = jnp.zeros((16,), jnp.int32)
        return carry

    lax.fori_loop(0, PASSES, _czero, 0)

    def _cnt_blk(blk, carry):
        off = pl.multiple_of(ebase + blk * EBLK, 8)
        pltpu.sync_copy(dst_hbm.at[pl.ds(off, EBLK)], dblk.at[pl.ds(0, EBLK)])

        def _cnt_vec(i, carry2):
            d = dblk[pl.ds(i * 16, 16)]
            rel = d - base
            b = lax.shift_right_arithmetic(rel, SHB)
            for bb in range(PASSES):
                cntb[pl.ds(bb * 16, 16)] = (
                    cntb[pl.ds(bb * 16, 16)] + jnp.where(b == bb, 1, 0))
            return carry2

        lax.fori_loop(0, EBLK // 16, _cnt_vec, 0)
        pltpu.sync_copy(dst_hbm.at[pl.ds(off + 984, 16)],
                        dblk.at[pl.ds(0, 16)])
        d = dblk[pl.ds(0, 16)]
        rel = d - base
        b = lax.shift_right_arithmetic(rel, SHB)
        mtail = lane >= 8
        for bb in range(PASSES):
            cntb[pl.ds(bb * 16, 16)] = (
                cntb[pl.ds(bb * 16, 16)]
                + jnp.where(jnp.logical_and(b == bb, mtail), 1, 0))
        return carry

    lax.fori_loop(0, NBLK, _cnt_blk, 0)
    cnt_s = [_vsum16(cntb[pl.ds(b * 16, 16)]) for b in range(PASSES)]
    seg = [jnp.int32(0)] * PASSES
    acc = jnp.int32(0)
    for b in range(PASSES):
        seg[b] = acc + 32 * b
        acc = acc + cnt_s[b]
    segv0 = jnp.zeros((16,), jnp.int32)
    segv1 = jnp.zeros((16,), jnp.int32)
    for b in range(16):
        segv0 = jnp.where(lane == b, seg[b], segv0)
    for b in range(16, PASSES):
        segv1 = jnp.where(lane == b - 16, seg[b], segv1)
    segbuf[pl.ds(0, 16)] = segv0
    segbuf[pl.ds(16, 16)] = segv1
    segbuf[pl.ds(32, 16)] = jnp.zeros((16,), jnp.int32)
    offbuf[pl.ds(0, 16)] = segv0
    offbuf[pl.ds(16, 16)] = segv1
    offbuf[pl.ds(32, 16)] = jnp.zeros((16,), jnp.int32)

    def _app_blk(blk, carry):
        off = pl.multiple_of(ebase + blk * EBLK, 8)
        pltpu.sync_copy(dst_hbm.at[pl.ds(off, EBLK)], dblk.at[pl.ds(0, EBLK)])
        pltpu.sync_copy(src_hbm.at[pl.ds(off, EBLK)], sblk.at[pl.ds(0, EBLK)])
        pltpu.sync_copy(attr_hbm.at[pl.ds(off, EBLK)],
                        ablk.at[pl.ds(0, EBLK)])

        def _app_edge(i, carry2):
            d = dblk[pl.ds(i, 16)][0]
            rel = d - base

            @pl.when(jnp.logical_and(rel >= 0, rel < HALF))
            def _append():
                b = lax.shift_right_arithmetic(rel, SHB)
                ov = offbuf[pl.ds(b, 16)]
                o = ov[0]
                packv = (
                    lax.shift_left(sblk[pl.ds(i, 16)], SHL)
                    | (dblk[pl.ds(i, 16)] - (base + b * C)))
                wl_pack[pl.ds(o, 16)] = packv
                wl_attr[pl.ds(o, 16)] = ablk[pl.ds(i, 16)]
                offbuf[pl.ds(b, 16)] = jnp.where(lane == 0, ov + 1, ov)

            return carry2

        lax.fori_loop(0, EBLK, _app_edge, 0, unroll=2)
        return carry

    lax.fori_loop(0, NBLK, _app_blk, 0)

    for b in range(PASSES):
        oe = offbuf[pl.ds(b, 16)][0]
        wl_pack[pl.ds(oe, 16)] = jnp.full((16,), C, jnp.int32)
        wl_attr[pl.ds(oe, 16)] = z16

    plsc.subcore_barrier()

    def _pass(p, carry0):
        sp = segbuf[pl.ds(p, 16)][0]
        ep = offbuf[pl.ds(p, 16)][0]
        n_chunks = lax.shift_right_logical(ep - sp + 15, 4)
        rowbase = base + p * C

        def _chunk(j, carry):
            wo = sp + j * 16
            packv = wl_pack[pl.ds(wo, 16)]
            srcv = lax.shift_right_logical(packv, SHL)
            dstlv = packv & (2 * C - 1)
            attv = wl_attr[pl.ds(wo, 16)]
            srcidx[pl.ds(0, 16)] = srcv
            qidx[pl.ds(0, 16)] = jnp.where(dstlv >= C, 0, rowbase + dstlv)
            dstidx[pl.ds(0, 16)] = dstlv
            abuf[pl.ds(0, 16)] = attv
            cpq = pltpu.async_copy(q_hbm.at[qidx], qbuf, semq)
            cpk = pltpu.async_copy(kv_hbm.at[srcidx], kvbuf, semk)
            cpq.wait()
            cpk.wait()

            def _edge(je, carry2):
                a = abuf[pl.ds(je, 16)][0]
                alpha = jnp.zeros((16,), jnp.float32)
                for h in range(HEADS):
                    acc_h = jnp.zeros((16,), jnp.float32)
                    for i in range(16):
                        cs = pl.ds(h * HEAD_DIM + i * 16, 16)
                        acc_h += qbuf[je, cs] * kvbuf[je, cs]
                    alpha = jnp.where(lane == h, _vsum16(acc_h), alpha)
                qev = qbuf[je, pl.ds(OUT_CH, 16)]
                alpha = (alpha + a * qev) * 0.0625
                ex = jnp.where(lane < HEADS, jnp.exp(alpha), 0.0)
                msg[8, je, pl.ds(0, 16)] = ex
                msg[8, je, pl.ds(16, 16)] = ex * a
                for h in range(HEADS):
                    sh = ex[h]
                    for i in range(16):
                        vs = pl.ds(OUT_CH + h * HEAD_DIM + i * 16, 16)
                        k2 = 2 * h + i // 8
                        msg[k2, je, pl.ds((i % 8) * 16, 16)] = (
                            sh * kvbuf[je, vs])
                return carry2

            lax.fori_loop(0, 16, _edge, 0, unroll=4)
            cps = [
                pltpu.async_copy(msg.at[k], numer_sh.at[k].at[dstidx],
                                 semm, add=True)
                for k in range(9)
            ]
            for cp in cps:
                cp.wait()
            return carry

        lax.fori_loop(0, n_chunks, _chunk, jnp.int32(0))
        plsc.subcore_barrier()

        def _fin(g, carry):
            r0 = pl.multiple_of(s * 16 + g * 8, 8)
            for k in range(9):
                pltpu.sync_copy(numer_sh.at[k, pl.ds(r0, 8)], rowbuf.at[k])

            def _row(r, carry2):
                dv = rowbuf[8, r, pl.ds(0, 16)]
                swav = rowbuf[8, r, pl.ds(16, 16)]
                dvi = jnp.full((16,), 1.0, jnp.float32) / (dv + 1e-16)
                for h in range(HEADS):
                    inv = dvi[h]
                    swa = swav[h]
                    for k2 in (2 * h, 2 * h + 1):
                        for i2 in range(8):
                            cs = pl.ds(i2 * 16, 16)
                            ws = pl.ds(k2 * 128 + i2 * 16, 16)
                            rowbuf[k2, r, cs] = (
                                rowbuf[k2, r, cs] + swa * webuf[ws]) * inv
                return carry2

            lax.fori_loop(0, 8, _row, 0, unroll=2)
            for k in range(8):
                pltpu.sync_copy(
                    rowbuf.at[k],
                    att_hbm.at[pl.ds(rowbase + r0, 8),
                               pl.ds(k * 128, 128)])
            for k in range(9):
                pltpu.sync_copy(zrow, numer_sh.at[k, pl.ds(r0, 8)])
            return carry

        lax.fori_loop(0, 2, _fin, 0)
        plsc.subcore_barrier()
        return carry0

    lax.fori_loop(0, PASSES, _pass, jnp.int32(0))


def _edge_attention(qx, kv, src, dst, attr, we):
    mesh = plsc.VectorSubcoreMesh(core_axis_name="c", subcore_axis_name="s")
    f = pl.kernel(
        _edge_body,
        mesh=mesh,
        out_type=jax.ShapeDtypeStruct((NPAD, OUT_CH), jnp.float32),
        scratch_types=[
            pltpu.VMEM((EBLK + 40,), jnp.int32),
            pltpu.VMEM((EBLK + 40,), jnp.int32),
            pltpu.VMEM((EBLK + 40,), jnp.float32),
            pltpu.VMEM((WCAP,), jnp.int32),
            pltpu.VMEM((WCAP,), jnp.float32),
            pltpu.VMEM((48,), jnp.int32),
            pltpu.VMEM((48,), jnp.int32),
            pltpu.VMEM((16 * PASSES,), jnp.int32),
            pltpu.VMEM((16, OUT_CH + 128), jnp.float32),
            pltpu.VMEM((16, 2 * OUT_CH), jnp.float32),
            pltpu.VMEM((9, 16, 128), jnp.float32),
            pltpu.VMEM((16,), jnp.int32),
            pltpu.VMEM((16,), jnp.int32),
            pltpu.VMEM((16,), jnp.int32),
            pltpu.VMEM((32,), jnp.float32),
            pltpu.VMEM((OUT_CH,), jnp.float32),
            pltpu.VMEM((9, 8, 128), jnp.float32),
            pltpu.VMEM((8, 128), jnp.float32),
            pltpu.VMEM_SHARED((9, CPAD, 128), jnp.float32),
            pltpu.SemaphoreType.DMA,
            pltpu.SemaphoreType.DMA,
            pltpu.SemaphoreType.DMA,
        ],
    )
    return f(qx, kv, src, dst, attr, we)


def kernel(x, edge_index, edge_attr, Wq, bq, Wk, bk, Wv, bv, We, Wskip,
           bskip, gn_weight, gn_bias, gn_mean_scale):
    Wcat = jnp.concatenate([Wq, Wk, Wv, Wskip], axis=1)
    bcat = jnp.concatenate([bq, bk, bv, bskip])[None, :]
    we = We.reshape(OUT_CH)
    proj, qe = _projections(x, Wcat, bcat, we[None, :])
    qx = jnp.concatenate([proj[:, 0 * OUT_CH:1 * OUT_CH], qe], axis=1)
    kv = proj[:, 1 * OUT_CH:3 * OUT_CH]
    skip = proj[:, 3 * OUT_CH:4 * OUT_CH]

    src = edge_index[0]
    dst = edge_index[1]
    attr = edge_attr.reshape(E)
    att = _edge_attention(qx, kv, src, dst, attr, we)

    out, s1, s2 = _stats(att[:N], skip)
    gram = _gram(out, s1, s2, gn_weight[None, :], gn_bias[None, :],
                 gn_mean_scale[None, :])
    iu = jnp.triu_indices(OUT_CH, k=1)
    return gram[iu].reshape(-1, 1)


# speedup vs baseline: 1.2304x; 1.2304x over previous
import functools

import jax
import jax.numpy as jnp
from jax import lax
from jax.experimental import pallas as pl
from jax.experimental.pallas import tpu as pltpu
from jax.experimental.pallas import tpu_sc as plsc

N = 10000
E = 160000
IN_CH = 256
OUT_CH = 1024
HEADS = 4
HEAD_DIM = OUT_CH // HEADS

NB = 512
GROWS = (N + NB - 1) // NB

C = 256
SHB = 8
SHL = 9
PASSES = 20
HALF = PASSES * C
NPAD = 2 * HALF
EPT = E // 16
CPAD = C + 16
ROW = OUT_CH + 128
EBLK = 1000
NBLK = EPT // EBLK
WCAP = EPT + PASSES * 32 + 32


def _proj_body(x_ref, w_ref, b_ref, we_ref, out_ref, qe_ref):
    o = (jnp.dot(x_ref[...], w_ref[...], preferred_element_type=jnp.float32)
         + b_ref[...])
    out_ref[...] = o
    qw = o[:, :OUT_CH] * we_ref[...]
    cols = [
        jnp.sum(qw[:, h * HEAD_DIM:(h + 1) * HEAD_DIM], axis=1,
                keepdims=True)
        for h in range(HEADS)
    ]
    cols.append(jnp.zeros((qw.shape[0], 128 - HEADS), jnp.float32))
    qe_ref[...] = jnp.concatenate(cols, axis=1)


def _projections(x, Wcat, bcat, we):
    return pl.pallas_call(
        _proj_body,
        grid=(GROWS,),
        in_specs=[
            pl.BlockSpec((NB, IN_CH), lambda i: (i, 0)),
            pl.BlockSpec((IN_CH, 4 * OUT_CH), lambda i: (0, 0)),
            pl.BlockSpec((1, 4 * OUT_CH), lambda i: (0, 0)),
            pl.BlockSpec((1, OUT_CH), lambda i: (0, 0)),
        ],
        out_specs=[
            pl.BlockSpec((NB, 4 * OUT_CH), lambda i: (i, 0)),
            pl.BlockSpec((NB, 128), lambda i: (i, 0)),
        ],
        out_shape=[
            jax.ShapeDtypeStruct((N, 4 * OUT_CH), jnp.float32),
            jax.ShapeDtypeStruct((N, 128), jnp.float32),
        ],
    )(x, Wcat, bcat, we)


def _stats_body(att_ref, skip_ref, out_ref, s1_ref, s2_ref):
    i = pl.program_id(0)
    o = att_ref[...] + skip_ref[...]
    out_ref[...] = o
    row = lax.broadcasted_iota(jnp.int32, o.shape, 0) + i * NB
    om = jnp.where(row < N, o, 0.0)
    s1 = jnp.sum(om, axis=0, keepdims=True)
    s2 = jnp.sum(om * om, axis=0, keepdims=True)

    @pl.when(i == 0)
    def _init():
        s1_ref[...] = jnp.zeros_like(s1_ref)
        s2_ref[...] = jnp.zeros_like(s2_ref)

    s1_ref[...] += s1
    s2_ref[...] += s2


def _stats(att, skip):
    return pl.pallas_call(
        _stats_body,
        grid=(GROWS,),
        in_specs=[
            pl.BlockSpec((NB, OUT_CH), lambda i: (i, 0)),
            pl.BlockSpec((NB, OUT_CH), lambda i: (i, 0)),
        ],
        out_specs=[
            pl.BlockSpec((NB, OUT_CH), lambda i: (i, 0)),
            pl.BlockSpec((1, OUT_CH), lambda i: (0, 0)),
            pl.BlockSpec((1, OUT_CH), lambda i: (0, 0)),
        ],
        out_shape=[
            jax.ShapeDtypeStruct((N, OUT_CH), jnp.float32),
            jax.ShapeDtypeStruct((1, OUT_CH), jnp.float32),
            jax.ShapeDtypeStruct((1, OUT_CH), jnp.float32),
        ],
    )(att, skip)


def _gram_body(out_ref, s1_ref, s2_ref, gw_ref, gb_ref, gms_ref, gram_ref):
    i = pl.program_id(0)
    mean = s1_ref[...] / N
    ex2 = s2_ref[...] / N
    g = gms_ref[...]
    var = ex2 - (2.0 * g - g * g) * mean * mean
    inv = lax.rsqrt(var + 1e-5)
    o = out_ref[...]
    onorm = gw_ref[...] * (o - g * mean) * inv + gb_ref[...]
    onorm = jnp.maximum(onorm, 0.0)
    row = lax.broadcasted_iota(jnp.int32, onorm.shape, 0) + i * NB
    onorm = jnp.where(row < N, onorm, 0.0)

    @pl.when(i == 0)
    def _init():
        gram_ref[...] = jnp.zeros_like(gram_ref)

    gram_ref[...] += lax.dot_general(
        onorm, onorm, (((0,), (0,)), ((), ())),
        preferred_element_type=jnp.float32,
    )


def _gram(out, s1, s2, gw, gb, gms):
    return pl.pallas_call(
        _gram_body,
        grid=(GROWS,),
        in_specs=[
            pl.BlockSpec((NB, OUT_CH), lambda i: (i, 0)),
            pl.BlockSpec((1, OUT_CH), lambda i: (0, 0)),
            pl.BlockSpec((1, OUT_CH), lambda i: (0, 0)),
            pl.BlockSpec((1, OUT_CH), lambda i: (0, 0)),
            pl.BlockSpec((1, OUT_CH), lambda i: (0, 0)),
            pl.BlockSpec((1, OUT_CH), lambda i: (0, 0)),
        ],
        out_specs=pl.BlockSpec((OUT_CH, OUT_CH), lambda i: (0, 0)),
        out_shape=jax.ShapeDtypeStruct((OUT_CH, OUT_CH), jnp.float32),
    )(out, s1, s2, gw, gb, gms)


def _vsum16(v):
    vals = [v[t] for t in range(16)]
    while len(vals) > 1:
        nxt = [vals[i] + vals[i + 1] for i in range(0, len(vals) - 1, 2)]
        if len(vals) % 2:
            nxt.append(vals[-1])
        vals = nxt
    return vals[0]


def _edge_body(q_hbm, kv_hbm, src_hbm, dst_hbm, attr_hbm, we_hbm,
               att_hbm, dblk, sblk, ablk, wl_pack, wl_attr, segbuf, offbuf,
               cntb, qbuf, kvbuf, msg, qidx, srcidx, dstidx, abuf,
               webuf, rowbuf, zrow, numer_sh, semq, semk, semm):
    c = lax.axis_index("c")
    s = lax.axis_index("s")
    lane = lax.broadcasted_iota(jnp.int32, (16,), 0)
    base = c * HALF
    ebase = s * EPT
    z16 = jnp.zeros((16,), jnp.float32)

    pltpu.sync_copy(we_hbm, webuf)

    def _zcol(i, carry3):
        def _zc2(r, carry4):
            zrow[r, pl.ds(i * 16, 16)] = z16
            return carry4
        lax.fori_loop(0, 8, _zc2, 0)
        return carry3

    lax.fori_loop(0, 128 // 16, _zcol, 0)

    def _mfill(r, carry2):
        def _mcol(i, carry3):
            msg[8, r, pl.ds(32 + i * 16, 16)] = z16
            return carry3
        lax.fori_loop(0, (128 - 32) // 16, _mcol, 0)
        return carry2

    lax.fori_loop(0, 16, _mfill, 0)

    for k in range(9):
        pltpu.sync_copy(zrow, numer_sh.at[k, pl.ds(s * 17, 8)])
        pltpu.sync_copy(zrow, numer_sh.at[k, pl.ds(s * 17 + 8, 8)])
        pltpu.sync_copy(zrow.at[pl.ds(0, 1)],
                        numer_sh.at[k, pl.ds(s * 17 + 16, 1)])

    def _czero(bb, carry):
        cntb[pl.ds(bb * 16, 16)] = jnp.zeros((16,), jnp.int32)
        return carry

    lax.fori_loop(0, PASSES, _czero, 0)

    def _cnt_blk(blk, carry):
        off = pl.multiple_of(ebase + blk * EBLK, 8)
        pltpu.sync_copy(dst_hbm.at[pl.ds(off, EBLK)], dblk.at[pl.ds(0, EBLK)])

        def _cnt_vec(i, carry2):
            d = dblk[pl.ds(i * 16, 16)]
            rel = d - base
            b = lax.shift_right_arithmetic(rel, SHB)
            for bb in range(PASSES):
                cntb[pl.ds(bb * 16, 16)] = (
                    cntb[pl.ds(bb * 16, 16)] + jnp.where(b == bb, 1, 0))
            return carry2

        lax.fori_loop(0, EBLK // 16, _cnt_vec, 0)
        pltpu.sync_copy(dst_hbm.at[pl.ds(off + 984, 16)],
                        dblk.at[pl.ds(0, 16)])
        d = dblk[pl.ds(0, 16)]
        rel = d - base
        b = lax.shift_right_arithmetic(rel, SHB)
        mtail = lane >= 8
        for bb in range(PASSES):
            cntb[pl.ds(bb * 16, 16)] = (
                cntb[pl.ds(bb * 16, 16)]
                + jnp.where(jnp.logical_and(b == bb, mtail), 1, 0))
        return carry

    lax.fori_loop(0, NBLK, _cnt_blk, 0)
    cnt_s = [_vsum16(cntb[pl.ds(b * 16, 16)]) for b in range(PASSES)]
    seg = [jnp.int32(0)] * PASSES
    acc = jnp.int32(0)
    for b in range(PASSES):
        seg[b] = acc + 32 * b
        acc = acc + cnt_s[b]
    segv0 = jnp.zeros((16,), jnp.int32)
    segv1 = jnp.zeros((16,), jnp.int32)
    for b in range(16):
        segv0 = jnp.where(lane == b, seg[b], segv0)
    for b in range(16, PASSES):
        segv1 = jnp.where(lane == b - 16, seg[b], segv1)
    segbuf[pl.ds(0, 16)] = segv0
    segbuf[pl.ds(16, 16)] = segv1
    segbuf[pl.ds(32, 16)] = jnp.zeros((16,), jnp.int32)
    offbuf[pl.ds(0, 16)] = segv0
    offbuf[pl.ds(16, 16)] = segv1
    offbuf[pl.ds(32, 16)] = jnp.zeros((16,), jnp.int32)

    def _app_blk(blk, carry):
        off = pl.multiple_of(ebase + blk * EBLK, 8)
        pltpu.sync_copy(dst_hbm.at[pl.ds(off, EBLK)], dblk.at[pl.ds(0, EBLK)])
        pltpu.sync_copy(src_hbm.at[pl.ds(off, EBLK)], sblk.at[pl.ds(0, EBLK)])
        pltpu.sync_copy(attr_hbm.at[pl.ds(off, EBLK)],
                        ablk.at[pl.ds(0, EBLK)])

        def _app_edge(i, carry2):
            d = dblk[pl.ds(i, 16)][0]
            rel = d - base

            @pl.when(jnp.logical_and(rel >= 0, rel < HALF))
            def _append():
                b = lax.shift_right_arithmetic(rel, SHB)
                ov = offbuf[pl.ds(b, 16)]
                o = ov[0]
                packv = (
                    lax.shift_left(sblk[pl.ds(i, 16)], SHL)
                    | (dblk[pl.ds(i, 16)] - (base + b * C)))
                wl_pack[pl.ds(o, 16)] = packv
                wl_attr[pl.ds(o, 16)] = ablk[pl.ds(i, 16)]
                offbuf[pl.ds(b, 16)] = jnp.where(lane == 0, ov + 1, ov)

            return carry2

        lax.fori_loop(0, EBLK, _app_edge, 0)
        return carry

    lax.fori_loop(0, NBLK, _app_blk, 0)

    for b in range(PASSES):
        oe = offbuf[pl.ds(b, 16)][0]
        wl_pack[pl.ds(oe, 16)] = jnp.full((16,), C, jnp.int32)
        wl_attr[pl.ds(oe, 16)] = z16

    plsc.subcore_barrier()

    def _pass(p, carry0):
        sp = segbuf[pl.ds(p, 16)][0]
        ep = offbuf[pl.ds(p, 16)][0]
        n_chunks = lax.shift_right_logical(ep - sp + 15, 4)
        rowbase = base + p * C

        def _chunk(j, carry):
            wo = sp + j * 16
            packv = wl_pack[pl.ds(wo, 16)]
            srcv = lax.shift_right_logical(packv, SHL)
            dstlv = packv & (2 * C - 1)
            attv = wl_attr[pl.ds(wo, 16)]
            srcidx[pl.ds(0, 16)] = srcv
            qidx[pl.ds(0, 16)] = jnp.where(dstlv >= C, 0, rowbase + dstlv)
            dstidx[pl.ds(0, 16)] = dstlv
            abuf[pl.ds(0, 16)] = attv
            cpq = pltpu.async_copy(q_hbm.at[qidx], qbuf, semq)
            cpk = pltpu.async_copy(kv_hbm.at[srcidx], kvbuf, semk)
            cpq.wait()
            cpk.wait()

            def _edge(je, carry2):
                a = abuf[pl.ds(je, 16)][0]
                alpha = jnp.zeros((16,), jnp.float32)
                for h in range(HEADS):
                    acc_h = jnp.zeros((16,), jnp.float32)
                    for i in range(16):
                        cs = pl.ds(h * HEAD_DIM + i * 16, 16)
                        acc_h += qbuf[je, cs] * kvbuf[je, cs]
                    alpha = jnp.where(lane == h, _vsum16(acc_h), alpha)
                qev = qbuf[je, pl.ds(OUT_CH, 16)]
                alpha = (alpha + a * qev) * 0.0625
                ex = jnp.where(lane < HEADS, jnp.exp(alpha), 0.0)
                msg[8, je, pl.ds(0, 16)] = ex
                msg[8, je, pl.ds(16, 16)] = ex * a
                for h in range(HEADS):
                    sh = ex[h]
                    for i in range(16):
                        vs = pl.ds(OUT_CH + h * HEAD_DIM + i * 16, 16)
                        k2 = 2 * h + i // 8
                        msg[k2, je, pl.ds((i % 8) * 16, 16)] = (
                            sh * kvbuf[je, vs])
                return carry2

            lax.fori_loop(0, 16, _edge, 0)
            cps = [
                pltpu.async_copy(msg.at[k], numer_sh.at[k].at[dstidx],
                                 semm, add=True)
                for k in range(9)
            ]
            for cp in cps:
                cp.wait()
            return carry

        lax.fori_loop(0, n_chunks, _chunk, jnp.int32(0))
        plsc.subcore_barrier()

        def _fin(g, carry):
            r0 = pl.multiple_of(s * 16 + g * 8, 8)
            for k in range(9):
                pltpu.sync_copy(numer_sh.at[k, pl.ds(r0, 8)], rowbuf.at[k])

            def _row(r, carry2):
                dv = rowbuf[8, r, pl.ds(0, 16)]
                swav = rowbuf[8, r, pl.ds(16, 16)]
                dvi = jnp.full((16,), 1.0, jnp.float32) / (dv + 1e-16)
                for h in range(HEADS):
                    inv = dvi[h]
                    swa = swav[h]
                    for k2 in (2 * h, 2 * h + 1):
                        for i2 in range(8):
                            cs = pl.ds(i2 * 16, 16)
                            ws = pl.ds(k2 * 128 + i2 * 16, 16)
                            rowbuf[k2, r, cs] = (
                                rowbuf[k2, r, cs] + swa * webuf[ws]) * inv
                return carry2

            lax.fori_loop(0, 8, _row, 0)
            for k in range(8):
                pltpu.sync_copy(
                    rowbuf.at[k],
                    att_hbm.at[pl.ds(rowbase + r0, 8),
                               pl.ds(k * 128, 128)])
            for k in range(9):
                pltpu.sync_copy(zrow, numer_sh.at[k, pl.ds(r0, 8)])
            return carry

        lax.fori_loop(0, 2, _fin, 0)
        plsc.subcore_barrier()
        return carry0

    lax.fori_loop(0, PASSES, _pass, jnp.int32(0))


def _edge_attention(qx, kv, src, dst, attr, we):
    mesh = plsc.VectorSubcoreMesh(core_axis_name="c", subcore_axis_name="s")
    f = pl.kernel(
        _edge_body,
        mesh=mesh,
        out_type=jax.ShapeDtypeStruct((NPAD, OUT_CH), jnp.float32),
        scratch_types=[
            pltpu.VMEM((EBLK + 40,), jnp.int32),
            pltpu.VMEM((EBLK + 40,), jnp.int32),
            pltpu.VMEM((EBLK + 40,), jnp.float32),
            pltpu.VMEM((WCAP,), jnp.int32),
            pltpu.VMEM((WCAP,), jnp.float32),
            pltpu.VMEM((48,), jnp.int32),
            pltpu.VMEM((48,), jnp.int32),
            pltpu.VMEM((16 * PASSES,), jnp.int32),
            pltpu.VMEM((16, OUT_CH + 128), jnp.float32),
            pltpu.VMEM((16, 2 * OUT_CH), jnp.float32),
            pltpu.VMEM((9, 16, 128), jnp.float32),
            pltpu.VMEM((16,), jnp.int32),
            pltpu.VMEM((16,), jnp.int32),
            pltpu.VMEM((16,), jnp.int32),
            pltpu.VMEM((32,), jnp.float32),
            pltpu.VMEM((OUT_CH,), jnp.float32),
            pltpu.VMEM((9, 8, 128), jnp.float32),
            pltpu.VMEM((8, 128), jnp.float32),
            pltpu.VMEM_SHARED((9, CPAD, 128), jnp.float32),
            pltpu.SemaphoreType.DMA,
            pltpu.SemaphoreType.DMA,
            pltpu.SemaphoreType.DMA,
        ],
    )
    return f(qx, kv, src, dst, attr, we)


def kernel(x, edge_index, edge_attr, Wq, bq, Wk, bk, Wv, bv, We, Wskip,
           bskip, gn_weight, gn_bias, gn_mean_scale):
    Wcat = jnp.concatenate([Wq, Wk, Wv, Wskip], axis=1)
    bcat = jnp.concatenate([bq, bk, bv, bskip])[None, :]
    we = We.reshape(OUT_CH)
    proj, qe = _projections(x, Wcat, bcat, we[None, :])
    qx = jnp.concatenate([proj[:, 0 * OUT_CH:1 * OUT_CH], qe], axis=1)
    kv = proj[:, 1 * OUT_CH:3 * OUT_CH]
    skip = proj[:, 3 * OUT_CH:4 * OUT_CH]

    src = edge_index[0]
    dst = edge_index[1]
    attr = edge_attr.reshape(E)
    att = _edge_attention(qx, kv, src, dst, attr, we)

    out, s1, s2 = _stats(att[:N], skip)
    gram = _gram(out, s1, s2, gn_weight[None, :], gn_bias[None, :],
                 gn_mean_scale[None, :])
    iu = jnp.triu_indices(OUT_CH, k=1)
    return gram[iu].reshape(-1, 1)
